# baseline (speedup 1.0000x reference)
import functools

import jax
import jax.numpy as jnp
from jax import lax
from jax.experimental import pallas as pl
from jax.experimental.pallas import tpu as pltpu
from jax.experimental.pallas import tpu_sc as plsc

N = 10000
NP = 10240
E = 160000
D = 256
HALF = 128
NSUB = 16
ROWS_PT = NP // NSUB

KA = 96
NCH = 112
EP = NSUB * KA * NCH
EROWS = EP // KA
NCH_D = EP // 32 // KA

_MESH = plsc.VectorSubcoreMesh(core_axis_name="c", subcore_axis_name="s")


@functools.partial(
    pl.kernel,
    out_type=jax.ShapeDtypeStruct((2 * NP, HALF), jnp.float32),
    mesh=_MESH,
    scratch_types=[
        pltpu.VMEM((NCH_D, KA), jnp.int32),
        pltpu.VMEM((KA,), jnp.int32),
        pltpu.VMEM((KA, HALF), jnp.float32),
        pltpu.VMEM_SHARED((NP, HALF), jnp.float32),
    ],
)
def _sc_deg(dstp_hbm, ones_hbm, zeros_hbm, out_hbm, dstb, dsti_v, ones_v, acc):
    c = lax.axis_index("c")
    s = lax.axis_index("s")
    pltpu.sync_copy(zeros_hbm, acc.at[pl.ds(s * ROWS_PT, ROWS_PT)])
    pltpu.sync_copy(ones_hbm, ones_v)
    pltpu.sync_copy(dstp_hbm.at[pl.ds((c * NSUB + s) * NCH_D, NCH_D)], dstb)
    plsc.subcore_barrier()

    def body(i, _):
        for j in range(KA // 16):
            sl = pl.ds(j * 16, 16)
            dsti_v[sl] = dstb[i, sl]
        pltpu.sync_copy(ones_v, acc.at[dsti_v], add=True)
        return _

    lax.fori_loop(0, NCH_D, body, None)
    plsc.subcore_barrier()
    pltpu.sync_copy(
        acc.at[pl.ds(s * ROWS_PT, ROWS_PT)],
        out_hbm.at[pl.ds(c * NP + s * ROWS_PT, ROWS_PT)],
    )


@functools.partial(
    pl.kernel,
    out_type=jax.ShapeDtypeStruct((2 * NP, HALF), jnp.float32),
    mesh=_MESH,
    scratch_types=[
        pltpu.VMEM((KA,), jnp.int32),
        pltpu.VMEM((KA,), jnp.int32),
        pltpu.VMEM((KA,), jnp.int32),
        pltpu.VMEM((KA,), jnp.int32),
        pltpu.VMEM((KA, HALF), jnp.float32),
        pltpu.VMEM((KA, HALF), jnp.float32),
        pltpu.SemaphoreType.DMA,
        pltpu.SemaphoreType.DMA,
        pltpu.SemaphoreType.DMA,
        pltpu.SemaphoreType.DMA,
        pltpu.VMEM_SHARED((NP, HALF), jnp.float32),
    ],
)
def _sc_agg(g_hbm, srcf_hbm, dstf_hbm, zeros_hbm, out_hbm,
            ix0, ix1, dx0, dx1, r0, r1, gs0, gs1, is0, is1, acc):
    ix = (ix0, ix1)
    dx = (dx0, dx1)
    rows = (r0, r1)
    gsem = (gs0, gs1)
    isem = (is0, is1)
    c = lax.axis_index("c")
    s = lax.axis_index("s")
    pltpu.sync_copy(zeros_hbm, acc.at[pl.ds(s * ROWS_PT, ROWS_PT)])
    tbase = s * NCH * KA

    plsc.subcore_barrier()

    def body(i, _):
        pltpu.sync_copy(srcf_hbm.at[pl.ds(c * EP + tbase + i * KA, KA)], ix0)
        pltpu.sync_copy(dstf_hbm.at[pl.ds(tbase + i * KA, KA)], dx0)
        pltpu.async_copy(g_hbm.at[ix0], r0, gs0).wait()
        pltpu.sync_copy(r0, acc.at[dx0], add=True)
        return _

    lax.fori_loop(0, NCH, body, None)
    plsc.subcore_barrier()
    pltpu.sync_copy(
        acc.at[pl.ds(s * ROWS_PT, ROWS_PT)],
        out_hbm.at[pl.ds(c * NP + s * ROWS_PT, ROWS_PT)],
    )


BM = 640
NB = NP // BM


def _dis_of(d0, d1):
    deg = d0[:, 0:1] + d1[:, 0:1] + 1.0
    return lax.rsqrt(deg)


def _tc_first_body(x_ref, w_ref, d0_ref, d1_ref, o_ref):
    dis = _dis_of(d0_ref[...], d1_ref[...])
    hw = jnp.dot(x_ref[...], w_ref[...], preferred_element_type=jnp.float32)
    o_ref[...] = hw * dis


def _tc_layer_body(s0, s1, g0, g1, d0, d1, b_ref, w_ref, o_ref):
    dis = _dis_of(d0[...], d1[...])
    h = jnp.concatenate([s0[...] + g0[...], s1[...] + g1[...]], axis=1)
    h = jnp.maximum(dis * h + b_ref[...], 0.0)
    o_ref[...] = jnp.dot(h, w_ref[...], preferred_element_type=jnp.float32) * dis


def _tc_cls_body(s0, s1, g0, g1, d0, d1, b_ref, wc_ref, bc_ref, o_ref):
    dis = _dis_of(d0[...], d1[...])
    h = jnp.concatenate([s0[...] + g0[...], s1[...] + g1[...]], axis=1)
    h = jnp.maximum(dis * h + b_ref[...], 0.0)
    logits = jnp.dot(h, wc_ref[...], preferred_element_type=jnp.float32) + bc_ref[...]
    m = jnp.max(logits, axis=1, keepdims=True)
    lse = m + jnp.log(jnp.sum(jnp.exp(logits - m), axis=1, keepdims=True))
    o_ref[...] = logits - lse


def _row_blk(c, i):
    return (i, 0)


def _row_blk_hi(c, i):
    return (NB + i, 0)


def _deg_specs():
    return [
        pl.BlockSpec((BM, HALF), _row_blk),
        pl.BlockSpec((BM, HALF), _row_blk_hi),
    ]


def _tc_first(x, w, degp):
    return pl.pallas_call(
        _tc_first_body,
        grid=(2, NB),
        in_specs=[
            pl.BlockSpec((BM, D), _row_blk),
            pl.BlockSpec((D, HALF), lambda c, i: (0, c)),
            *_deg_specs(),
        ],
        out_specs=pl.BlockSpec((BM, HALF), lambda c, i: (c * NB + i, 0)),
        out_shape=jax.ShapeDtypeStruct((2 * NP, HALF), jnp.float32),
    )(x, w, degp, degp)


def _tc_layer(scat, gcat, degp, b2d, w):
    return pl.pallas_call(
        _tc_layer_body,
        grid=(2, NB),
        in_specs=[
            pl.BlockSpec((BM, HALF), _row_blk),
            pl.BlockSpec((BM, HALF), _row_blk_hi),
            pl.BlockSpec((BM, HALF), _row_blk),
            pl.BlockSpec((BM, HALF), _row_blk_hi),
            *_deg_specs(),
            pl.BlockSpec((1, D), lambda c, i: (0, 0)),
            pl.BlockSpec((D, HALF), lambda c, i: (0, c)),
        ],
        out_specs=pl.BlockSpec((BM, HALF), lambda c, i: (c * NB + i, 0)),
        out_shape=jax.ShapeDtypeStruct((2 * NP, HALF), jnp.float32),
    )(scat, scat, gcat, gcat, degp, degp, b2d, w)


def _tc_cls(scat, gcat, degp, b2d, wcp, bcp):
    return pl.pallas_call(
        _tc_cls_body,
        grid=(NB,),
        in_specs=[
            pl.BlockSpec((BM, HALF), lambda i: (i, 0)),
            pl.BlockSpec((BM, HALF), lambda i: (NB + i, 0)),
            pl.BlockSpec((BM, HALF), lambda i: (i, 0)),
            pl.BlockSpec((BM, HALF), lambda i: (NB + i, 0)),
            pl.BlockSpec((BM, HALF), lambda i: (i, 0)),
            pl.BlockSpec((BM, HALF), lambda i: (NB + i, 0)),
            pl.BlockSpec((1, D), lambda i: (0, 0)),
            pl.BlockSpec((D, HALF), lambda i: (0, 0)),
            pl.BlockSpec((1, HALF), lambda i: (0, 0)),
        ],
        out_specs=pl.BlockSpec((BM, HALF), lambda i: (i, 0)),
        out_shape=jax.ShapeDtypeStruct((NP, HALF), jnp.float32),
    )(scat, scat, gcat, gcat, degp, degp, b2d, wcp, bcp)


def kernel(x, edge_index, W1, b1, W2, b2, W3, b3, Wc, bc):
    src = edge_index[0].astype(jnp.int32)
    dst = edge_index[1].astype(jnp.int32)
    pad = EP - E
    srcp = jnp.concatenate([src, jnp.zeros((pad,), jnp.int32)])
    dstpad = jnp.concatenate([dst, jnp.full((pad,), NP - 1, jnp.int32)])
    srcf = jnp.concatenate([srcp, srcp + NP])
    dstp = dstpad.reshape(EROWS, KA)
    ones_ka = jnp.ones((KA, HALF), jnp.float32)
    zeros128 = jnp.zeros((ROWS_PT, HALF), jnp.float32)
    wcp = jnp.zeros((D, HALF), jnp.float32).at[:, :2].set(Wc)
    bcp = jnp.full((1, HALF), -1e30, jnp.float32).at[0, :2].set(bc)
    x_p = jnp.zeros((NP, D), jnp.float32).at[:N].set(x)

    degp = _sc_deg(dstp, ones_ka, zeros128)
    g = _tc_first(x_p, W1, degp)
    s = _sc_agg(g, srcf, dstpad, zeros128)
    g = _tc_layer(s, g, degp, b1.reshape(1, D), W2)
    s = _sc_agg(g, srcf, dstpad, zeros128)
    g = _tc_layer(s, g, degp, b2.reshape(1, D), W3)
    s = _sc_agg(g, srcf, dstpad, zeros128)
    outp = _tc_cls(s, g, degp, b3.reshape(1, D), wcp, bcp)
    return outp[:N, :2]


# speedup vs baseline: 1.7247x; 1.7247x over previous
import functools

import jax
import jax.numpy as jnp
from jax import lax
from jax.experimental import pallas as pl
from jax.experimental.pallas import tpu as pltpu
from jax.experimental.pallas import tpu_sc as plsc

N = 10000
NP = 10240
E = 160000
D = 256
HALF = 128
NSUB = 16
ROWS_PT = NP // NSUB

KA = 80
NCH = 128
EP = NSUB * KA * NCH
EROWS = EP // KA
NCH_D = EP // 32 // KA

_MESH = plsc.VectorSubcoreMesh(core_axis_name="c", subcore_axis_name="s")


@functools.partial(
    pl.kernel,
    out_type=jax.ShapeDtypeStruct((2 * NP, HALF), jnp.float32),
    mesh=_MESH,
    scratch_types=[
        pltpu.VMEM((NCH_D, KA), jnp.int32),
        pltpu.VMEM((KA,), jnp.int32),
        pltpu.VMEM((KA, HALF), jnp.float32),
        pltpu.VMEM_SHARED((NP, HALF), jnp.float32),
    ],
)
def _sc_deg(dstp_hbm, ones_hbm, zeros_hbm, out_hbm, dstb, dsti_v, ones_v, acc):
    c = lax.axis_index("c")
    s = lax.axis_index("s")
    pltpu.sync_copy(zeros_hbm, acc.at[pl.ds(s * ROWS_PT, ROWS_PT)])
    pltpu.sync_copy(ones_hbm, ones_v)
    pltpu.sync_copy(dstp_hbm.at[pl.ds((c * NSUB + s) * NCH_D, NCH_D)], dstb)
    plsc.subcore_barrier()

    def body(i, _):
        for j in range(KA // 16):
            sl = pl.ds(j * 16, 16)
            dsti_v[sl] = dstb[i, sl]
        pltpu.sync_copy(ones_v, acc.at[dsti_v], add=True)
        return _

    lax.fori_loop(0, NCH_D, body, None)
    plsc.subcore_barrier()
    pltpu.sync_copy(
        acc.at[pl.ds(s * ROWS_PT, ROWS_PT)],
        out_hbm.at[pl.ds(c * NP + s * ROWS_PT, ROWS_PT)],
    )


@functools.partial(
    pl.kernel,
    out_type=jax.ShapeDtypeStruct((2 * NP, HALF), jnp.float32),
    mesh=_MESH,
    scratch_types=[
        pltpu.VMEM((KA,), jnp.int32),
        pltpu.VMEM((KA,), jnp.int32),
        pltpu.VMEM((KA,), jnp.int32),
        pltpu.VMEM((KA,), jnp.int32),
        pltpu.VMEM((KA, HALF), jnp.float32),
        pltpu.VMEM((KA, HALF), jnp.float32),
        pltpu.SemaphoreType.DMA,
        pltpu.SemaphoreType.DMA,
        pltpu.SemaphoreType.DMA,
        pltpu.SemaphoreType.DMA,
        pltpu.VMEM_SHARED((NP, HALF), jnp.float32),
    ],
)
def _sc_agg(g_hbm, srcf_hbm, dstf_hbm, zeros_hbm, out_hbm,
            ix0, ix1, dx0, dx1, r0, r1, gs0, gs1, is0, is1, acc):
    ix = (ix0, ix1)
    dx = (dx0, dx1)
    rows = (r0, r1)
    gsem = (gs0, gs1)
    isem = (is0, is1)
    c = lax.axis_index("c")
    s = lax.axis_index("s")
    pltpu.sync_copy(zeros_hbm, acc.at[pl.ds(s * ROWS_PT, ROWS_PT)])
    tbase = s * NCH * KA

    plsc.subcore_barrier()

    def body(i, _):
        pltpu.sync_copy(srcf_hbm.at[pl.ds(c * EP + tbase + i * KA, KA)], ix0)
        pltpu.sync_copy(dstf_hbm.at[pl.ds(tbase + i * KA, KA)], dx0)
        pltpu.async_copy(g_hbm.at[ix0], r0, gs0).wait()
        pltpu.sync_copy(r0, acc.at[dx0], add=True)
        return _

    lax.fori_loop(0, NCH, body, None)
    plsc.subcore_barrier()
    pltpu.sync_copy(
        acc.at[pl.ds(s * ROWS_PT, ROWS_PT)],
        out_hbm.at[pl.ds(c * NP + s * ROWS_PT, ROWS_PT)],
    )


BM = 640
NB = NP // BM


def _dis_of(d0, d1):
    deg = d0[:, 0:1] + d1[:, 0:1] + 1.0
    return lax.rsqrt(deg)


def _tc_first_body(x_ref, w_ref, d0_ref, d1_ref, o_ref):
    dis = _dis_of(d0_ref[...], d1_ref[...])
    hw = jnp.dot(x_ref[...], w_ref[...], preferred_element_type=jnp.float32)
    o_ref[...] = hw * dis


def _tc_layer_body(s0, s1, g0, g1, d0, d1, b_ref, w_ref, o_ref):
    dis = _dis_of(d0[...], d1[...])
    h = jnp.concatenate([s0[...] + g0[...], s1[...] + g1[...]], axis=1)
    h = jnp.maximum(dis * h + b_ref[...], 0.0)
    o_ref[...] = jnp.dot(h, w_ref[...], preferred_element_type=jnp.float32) * dis


def _tc_cls_body(s0, s1, g0, g1, d0, d1, b_ref, wc_ref, bc_ref, o_ref):
    dis = _dis_of(d0[...], d1[...])
    h = jnp.concatenate([s0[...] + g0[...], s1[...] + g1[...]], axis=1)
    h = jnp.maximum(dis * h + b_ref[...], 0.0)
    logits = jnp.dot(h, wc_ref[...], preferred_element_type=jnp.float32) + bc_ref[...]
    m = jnp.max(logits, axis=1, keepdims=True)
    lse = m + jnp.log(jnp.sum(jnp.exp(logits - m), axis=1, keepdims=True))
    o_ref[...] = logits - lse


def _row_blk(c, i):
    return (i, 0)


def _row_blk_hi(c, i):
    return (NB + i, 0)


def _deg_specs():
    return [
        pl.BlockSpec((BM, HALF), _row_blk),
        pl.BlockSpec((BM, HALF), _row_blk_hi),
    ]


def _tc_first(x, w, degp):
    return pl.pallas_call(
        _tc_first_body,
        grid=(2, NB),
        in_specs=[
            pl.BlockSpec((BM, D), _row_blk),
            pl.BlockSpec((D, HALF), lambda c, i: (0, c)),
            *_deg_specs(),
        ],
        out_specs=pl.BlockSpec((BM, HALF), lambda c, i: (c * NB + i, 0)),
        out_shape=jax.ShapeDtypeStruct((2 * NP, HALF), jnp.float32),
    )(x, w, degp, degp)


def _tc_layer(scat, gcat, degp, b2d, w):
    return pl.pallas_call(
        _tc_layer_body,
        grid=(2, NB),
        in_specs=[
            pl.BlockSpec((BM, HALF), _row_blk),
            pl.BlockSpec((BM, HALF), _row_blk_hi),
            pl.BlockSpec((BM, HALF), _row_blk),
            pl.BlockSpec((BM, HALF), _row_blk_hi),
            *_deg_specs(),
            pl.BlockSpec((1, D), lambda c, i: (0, 0)),
            pl.BlockSpec((D, HALF), lambda c, i: (0, c)),
        ],
        out_specs=pl.BlockSpec((BM, HALF), lambda c, i: (c * NB + i, 0)),
        out_shape=jax.ShapeDtypeStruct((2 * NP, HALF), jnp.float32),
    )(scat, scat, gcat, gcat, degp, degp, b2d, w)


def _tc_cls(scat, gcat, degp, b2d, wcp, bcp):
    return pl.pallas_call(
        _tc_cls_body,
        grid=(NB,),
        in_specs=[
            pl.BlockSpec((BM, HALF), lambda i: (i, 0)),
            pl.BlockSpec((BM, HALF), lambda i: (NB + i, 0)),
            pl.BlockSpec((BM, HALF), lambda i: (i, 0)),
            pl.BlockSpec((BM, HALF), lambda i: (NB + i, 0)),
            pl.BlockSpec((BM, HALF), lambda i: (i, 0)),
            pl.BlockSpec((BM, HALF), lambda i: (NB + i, 0)),
            pl.BlockSpec((1, D), lambda i: (0, 0)),
            pl.BlockSpec((D, HALF), lambda i: (0, 0)),
            pl.BlockSpec((1, HALF), lambda i: (0, 0)),
        ],
        out_specs=pl.BlockSpec((BM, HALF), lambda i: (i, 0)),
        out_shape=jax.ShapeDtypeStruct((NP, HALF), jnp.float32),
    )(scat, scat, gcat, gcat, degp, degp, b2d, wcp, bcp)


def kernel(x, edge_index, W1, b1, W2, b2, W3, b3, Wc, bc):
    src = edge_index[0].astype(jnp.int32)
    dst = edge_index[1].astype(jnp.int32)
    pad = EP - E
    srcp = jnp.concatenate([src, jnp.zeros((pad,), jnp.int32)])
    dstpad = jnp.concatenate([dst, jnp.full((pad,), NP - 1, jnp.int32)])
    srcf = jnp.concatenate([srcp, srcp + NP])
    dstp = dstpad.reshape(EROWS, KA)
    ones_ka = jnp.ones((KA, HALF), jnp.float32)
    zeros128 = jnp.zeros((ROWS_PT, HALF), jnp.float32)
    wcp = jnp.zeros((D, HALF), jnp.float32).at[:, :2].set(Wc)
    bcp = jnp.full((1, HALF), -1e30, jnp.float32).at[0, :2].set(bc)
    x_p = jnp.zeros((NP, D), jnp.float32).at[:N].set(x)

    degp = _sc_deg(dstp, ones_ka, zeros128)
    g = _tc_first(x_p, W1, degp)
    s = _sc_agg(g, srcf, dstpad, zeros128)
    g = _tc_layer(s, g, degp, b1.reshape(1, D), W2)
    s = _sc_agg(g, srcf, dstpad, zeros128)
    g = _tc_layer(s, g, degp, b2.reshape(1, D), W3)
    s = _sc_agg(g, srcf, dstpad, zeros128)
    outp = _tc_cls(s, g, degp, b3.reshape(1, D), wcp, bcp)
    return outp[:N, :2]


# speedup vs baseline: 2.6075x; 1.5119x over previous
import functools

import jax
import jax.numpy as jnp
from jax import lax
from jax.experimental import pallas as pl
from jax.experimental.pallas import tpu as pltpu
from jax.experimental.pallas import tpu_sc as plsc

N = 10000
NP = 10240
E = 160000
D = 256
HALF = 128
NSUB = 16
ROWS_PT = NP // NSUB

KA = 80
NCH = 128
EP = NSUB * KA * NCH
EROWS = EP // KA
NCH_D = EP // 32 // KA

_MESH = plsc.VectorSubcoreMesh(core_axis_name="c", subcore_axis_name="s")


@functools.partial(
    pl.kernel,
    out_type=jax.ShapeDtypeStruct((2 * NP, HALF), jnp.float32),
    mesh=_MESH,
    scratch_types=[
        pltpu.VMEM((NCH_D, KA), jnp.int32),
        pltpu.VMEM((KA,), jnp.int32),
        pltpu.VMEM((KA, HALF), jnp.float32),
        pltpu.VMEM_SHARED((NP, HALF), jnp.float32),
    ],
)
def _sc_deg(dstp_hbm, ones_hbm, zeros_hbm, out_hbm, dstb, dsti_v, ones_v, acc):
    c = lax.axis_index("c")
    s = lax.axis_index("s")
    pltpu.sync_copy(zeros_hbm, acc.at[pl.ds(s * ROWS_PT, ROWS_PT)])
    pltpu.sync_copy(ones_hbm, ones_v)
    pltpu.sync_copy(dstp_hbm.at[pl.ds((c * NSUB + s) * NCH_D, NCH_D)], dstb)
    plsc.subcore_barrier()

    def body(i, _):
        for j in range(KA // 16):
            sl = pl.ds(j * 16, 16)
            dsti_v[sl] = dstb[i, sl]
        pltpu.sync_copy(ones_v, acc.at[dsti_v], add=True)
        return _

    lax.fori_loop(0, NCH_D, body, None)
    plsc.subcore_barrier()
    pltpu.sync_copy(
        acc.at[pl.ds(s * ROWS_PT, ROWS_PT)],
        out_hbm.at[pl.ds(c * NP + s * ROWS_PT, ROWS_PT)],
    )


ECHT = E // NSUB
NAGG = ECHT // KA


@functools.partial(
    pl.kernel,
    out_type=jax.ShapeDtypeStruct((2 * NP, HALF), jnp.float32),
    mesh=_MESH,
    scratch_types=[
        pltpu.VMEM((KA,), jnp.int32),
        pltpu.VMEM((KA,), jnp.int32),
        pltpu.VMEM((KA, HALF), jnp.float32),
        pltpu.SemaphoreType.DMA,
        pltpu.VMEM_SHARED((NP, HALF), jnp.float32),
    ],
)
def _sc_agg(g_hbm, srcf_hbm, dstf_hbm, zeros_hbm, out_hbm,
            ix0, dx0, r0, gs0, acc):
    c = lax.axis_index("c")
    s = lax.axis_index("s")
    pltpu.sync_copy(zeros_hbm, acc.at[pl.ds(s * ROWS_PT, ROWS_PT)])
    tbase = s * ECHT
    plsc.subcore_barrier()

    def body(i, _):
        pltpu.sync_copy(srcf_hbm.at[pl.ds(c * E + tbase + i * KA, KA)], ix0)
        pltpu.sync_copy(dstf_hbm.at[pl.ds(tbase + i * KA, KA)], dx0)
        pltpu.async_copy(g_hbm.at[ix0], r0, gs0).wait()
        pltpu.sync_copy(r0, acc.at[dx0], add=True)
        return _

    lax.fori_loop(0, NAGG, body, None)
    plsc.subcore_barrier()
    pltpu.sync_copy(
        acc.at[pl.ds(s * ROWS_PT, ROWS_PT)],
        out_hbm.at[pl.ds(c * NP + s * ROWS_PT, ROWS_PT)],
    )


BM = 640
NB = NP // BM


def _dis_of(d0, d1):
    deg = d0[:, 0:1] + d1[:, 0:1] + 1.0
    return lax.rsqrt(deg)


def _tc_first_body(x_ref, w_ref, d0_ref, d1_ref, o_ref):
    dis = _dis_of(d0_ref[...], d1_ref[...])
    hw = jnp.dot(x_ref[...], w_ref[...], preferred_element_type=jnp.float32)
    o_ref[...] = hw * dis


def _tc_layer_body(s0, s1, g0, g1, d0, d1, b_ref, w_ref, o_ref):
    dis = _dis_of(d0[...], d1[...])
    h = jnp.concatenate([s0[...] + g0[...], s1[...] + g1[...]], axis=1)
    h = jnp.maximum(dis * h + b_ref[...], 0.0)
    o_ref[...] = jnp.dot(h, w_ref[...], preferred_element_type=jnp.float32) * dis


def _tc_cls_body(s0, s1, g0, g1, d0, d1, b_ref, wc_ref, bc_ref, o_ref):
    dis = _dis_of(d0[...], d1[...])
    h = jnp.concatenate([s0[...] + g0[...], s1[...] + g1[...]], axis=1)
    h = jnp.maximum(dis * h + b_ref[...], 0.0)
    logits = jnp.dot(h, wc_ref[...], preferred_element_type=jnp.float32) + bc_ref[...]
    m = jnp.max(logits, axis=1, keepdims=True)
    lse = m + jnp.log(jnp.sum(jnp.exp(logits - m), axis=1, keepdims=True))
    o_ref[...] = logits - lse


def _row_blk(c, i):
    return (i, 0)


def _row_blk_hi(c, i):
    return (NB + i, 0)


def _deg_specs():
    return [
        pl.BlockSpec((BM, HALF), _row_blk),
        pl.BlockSpec((BM, HALF), _row_blk_hi),
    ]


def _tc_first(x, w, degp):
    return pl.pallas_call(
        _tc_first_body,
        grid=(2, NB),
        in_specs=[
            pl.BlockSpec((BM, D), _row_blk),
            pl.BlockSpec((D, HALF), lambda c, i: (0, c)),
            *_deg_specs(),
        ],
        out_specs=pl.BlockSpec((BM, HALF), lambda c, i: (c * NB + i, 0)),
        out_shape=jax.ShapeDtypeStruct((2 * NP, HALF), jnp.float32),
    )(x, w, degp, degp)


def _tc_layer(scat, gcat, degp, b2d, w):
    return pl.pallas_call(
        _tc_layer_body,
        grid=(2, NB),
        in_specs=[
            pl.BlockSpec((BM, HALF), _row_blk),
            pl.BlockSpec((BM, HALF), _row_blk_hi),
            pl.BlockSpec((BM, HALF), _row_blk),
            pl.BlockSpec((BM, HALF), _row_blk_hi),
            *_deg_specs(),
            pl.BlockSpec((1, D), lambda c, i: (0, 0)),
            pl.BlockSpec((D, HALF), lambda c, i: (0, c)),
        ],
        out_specs=pl.BlockSpec((BM, HALF), lambda c, i: (c * NB + i, 0)),
        out_shape=jax.ShapeDtypeStruct((2 * NP, HALF), jnp.float32),
    )(scat, scat, gcat, gcat, degp, degp, b2d, w)


def _tc_cls(scat, gcat, degp, b2d, wcp, bcp):
    return pl.pallas_call(
        _tc_cls_body,
        grid=(NB,),
        in_specs=[
            pl.BlockSpec((BM, HALF), lambda i: (i, 0)),
            pl.BlockSpec((BM, HALF), lambda i: (NB + i, 0)),
            pl.BlockSpec((BM, HALF), lambda i: (i, 0)),
            pl.BlockSpec((BM, HALF), lambda i: (NB + i, 0)),
            pl.BlockSpec((BM, HALF), lambda i: (i, 0)),
            pl.BlockSpec((BM, HALF), lambda i: (NB + i, 0)),
            pl.BlockSpec((1, D), lambda i: (0, 0)),
            pl.BlockSpec((D, HALF), lambda i: (0, 0)),
            pl.BlockSpec((1, HALF), lambda i: (0, 0)),
        ],
        out_specs=pl.BlockSpec((BM, HALF), lambda i: (i, 0)),
        out_shape=jax.ShapeDtypeStruct((NP, HALF), jnp.float32),
    )(scat, scat, gcat, gcat, degp, degp, b2d, wcp, bcp)


def kernel(x, edge_index, W1, b1, W2, b2, W3, b3, Wc, bc):
    src = edge_index[0].astype(jnp.int32)
    dst = edge_index[1].astype(jnp.int32)
    pad = EP - E
    srcp = jnp.concatenate([src, jnp.zeros((pad,), jnp.int32)])
    dstpad = jnp.concatenate([dst, jnp.full((pad,), NP - 1, jnp.int32)])
    srcf = jnp.concatenate([src, src + NP])
    dstp = dstpad.reshape(EROWS, KA)
    ones_ka = jnp.ones((KA, HALF), jnp.float32)
    zeros128 = jnp.zeros((ROWS_PT, HALF), jnp.float32)
    wcp = jnp.zeros((D, HALF), jnp.float32).at[:, :2].set(Wc)
    bcp = jnp.full((1, HALF), -1e30, jnp.float32).at[0, :2].set(bc)
    x_p = jnp.zeros((NP, D), jnp.float32).at[:N].set(x)

    degp = _sc_deg(dstp, ones_ka, zeros128)
    g = _tc_first(x_p, W1, degp)
    s = _sc_agg(g, srcf, dst, zeros128)
    g = _tc_layer(s, g, degp, b1.reshape(1, D), W2)
    s = _sc_agg(g, srcf, dst, zeros128)
    g = _tc_layer(s, g, degp, b2.reshape(1, D), W3)
    s = _sc_agg(g, srcf, dst, zeros128)
    outp = _tc_cls(s, g, degp, b3.reshape(1, D), wcp, bcp)
    return outp[:N, :2]


# speedup vs baseline: 2.6081x; 1.0002x over previous
import functools

import jax
import jax.numpy as jnp
from jax import lax
from jax.experimental import pallas as pl
from jax.experimental.pallas import tpu as pltpu
from jax.experimental.pallas import tpu_sc as plsc

N = 10000
NP = 10240
E = 160000
D = 256
HALF = 128
NSUB = 16
ROWS_PT = NP // NSUB

KA = 80
EP = 163840
EROWS = EP // KA
NCH_D = EP // 32 // KA

_MESH = plsc.VectorSubcoreMesh(core_axis_name="c", subcore_axis_name="s")


@functools.partial(
    pl.kernel,
    out_type=jax.ShapeDtypeStruct((2 * NP, HALF), jnp.float32),
    mesh=_MESH,
    scratch_types=[
        pltpu.VMEM((NCH_D, KA), jnp.int32),
        pltpu.VMEM((KA,), jnp.int32),
        pltpu.VMEM((KA, HALF), jnp.float32),
        pltpu.VMEM_SHARED((NP, HALF), jnp.float32),
    ],
)
def _sc_deg(dstp_hbm, ones_hbm, zeros_hbm, out_hbm, dstb, dsti_v, ones_v, acc):
    c = lax.axis_index("c")
    s = lax.axis_index("s")
    pltpu.sync_copy(zeros_hbm, acc.at[pl.ds(s * ROWS_PT, ROWS_PT)])
    pltpu.sync_copy(ones_hbm, ones_v)
    pltpu.sync_copy(dstp_hbm.at[pl.ds((c * NSUB + s) * NCH_D, NCH_D)], dstb)
    plsc.subcore_barrier()

    def body(i, _):
        for j in range(KA // 16):
            sl = pl.ds(j * 16, 16)
            dsti_v[sl] = dstb[i, sl]
        pltpu.sync_copy(ones_v, acc.at[dsti_v], add=True)
        return _

    lax.fori_loop(0, NCH_D, body, None)
    plsc.subcore_barrier()
    pltpu.sync_copy(
        acc.at[pl.ds(s * ROWS_PT, ROWS_PT)],
        out_hbm.at[pl.ds(c * NP + s * ROWS_PT, ROWS_PT)],
    )


ECHT = E // NSUB
NAGG = ECHT // KA


@functools.partial(
    pl.kernel,
    out_type=jax.ShapeDtypeStruct((2 * NP, HALF), jnp.float32),
    mesh=_MESH,
    scratch_types=[
        pltpu.VMEM((KA,), jnp.int32),
        pltpu.VMEM((KA,), jnp.int32),
        pltpu.VMEM((KA, HALF), jnp.float32),
        pltpu.SemaphoreType.DMA,
        pltpu.VMEM_SHARED((NP, HALF), jnp.float32),
    ],
)
def _sc_agg(g_hbm, srcf_hbm, dstf_hbm, zeros_hbm, out_hbm,
            ix0, dx0, r0, gs0, acc):
    c = lax.axis_index("c")
    s = lax.axis_index("s")
    pltpu.sync_copy(zeros_hbm, acc.at[pl.ds(s * ROWS_PT, ROWS_PT)])
    tbase = s * ECHT
    plsc.subcore_barrier()

    def body(i, _):
        pltpu.sync_copy(srcf_hbm.at[pl.ds(c * E + tbase + i * KA, KA)], ix0)
        pltpu.sync_copy(dstf_hbm.at[pl.ds(tbase + i * KA, KA)], dx0)
        pltpu.async_copy(g_hbm.at[ix0], r0, gs0).wait()
        pltpu.sync_copy(r0, acc.at[dx0], add=True)
        return _

    lax.fori_loop(0, NAGG, body, None)
    plsc.subcore_barrier()
    pltpu.sync_copy(
        acc.at[pl.ds(s * ROWS_PT, ROWS_PT)],
        out_hbm.at[pl.ds(c * NP + s * ROWS_PT, ROWS_PT)],
    )


BM = 640
NB = NP // BM


def _dis_of(d0, d1):
    deg = d0[:, 0:1] + d1[:, 0:1] + 1.0
    return lax.rsqrt(deg)


def _tc_first_body(x_ref, w_ref, d0_ref, d1_ref, o_ref):
    dis = _dis_of(d0_ref[...], d1_ref[...])
    hw = jnp.dot(x_ref[...], w_ref[...], preferred_element_type=jnp.float32)
    o_ref[...] = hw * dis


def _tc_layer_body(s0, s1, g0, g1, d0, d1, b_ref, w_ref, o_ref):
    dis = _dis_of(d0[...], d1[...])
    h = jnp.concatenate([s0[...] + g0[...], s1[...] + g1[...]], axis=1)
    h = jnp.maximum(dis * h + b_ref[...], 0.0)
    o_ref[...] = jnp.dot(h, w_ref[...], preferred_element_type=jnp.float32) * dis


def _tc_cls_body(s0, s1, g0, g1, d0, d1, b_ref, wc_ref, bc_ref, o_ref):
    dis = _dis_of(d0[...], d1[...])
    h = jnp.concatenate([s0[...] + g0[...], s1[...] + g1[...]], axis=1)
    h = jnp.maximum(dis * h + b_ref[...], 0.0)
    logits = jnp.dot(h, wc_ref[...], preferred_element_type=jnp.float32) + bc_ref[...]
    m = jnp.max(logits, axis=1, keepdims=True)
    lse = m + jnp.log(jnp.sum(jnp.exp(logits - m), axis=1, keepdims=True))
    o_ref[...] = logits - lse


def _row_blk(c, i):
    return (i, 0)


def _row_blk_hi(c, i):
    return (NB + i, 0)


def _deg_specs():
    return [
        pl.BlockSpec((BM, HALF), _row_blk),
        pl.BlockSpec((BM, HALF), _row_blk_hi),
    ]


def _tc_first(x, w, degp):
    return pl.pallas_call(
        _tc_first_body,
        grid=(2, NB),
        in_specs=[
            pl.BlockSpec((BM, D), _row_blk),
            pl.BlockSpec((D, HALF), lambda c, i: (0, c)),
            *_deg_specs(),
        ],
        out_specs=pl.BlockSpec((BM, HALF), lambda c, i: (c * NB + i, 0)),
        out_shape=jax.ShapeDtypeStruct((2 * NP, HALF), jnp.float32),
    )(x, w, degp, degp)


def _tc_layer(scat, gcat, degp, b2d, w):
    return pl.pallas_call(
        _tc_layer_body,
        grid=(2, NB),
        in_specs=[
            pl.BlockSpec((BM, HALF), _row_blk),
            pl.BlockSpec((BM, HALF), _row_blk_hi),
            pl.BlockSpec((BM, HALF), _row_blk),
            pl.BlockSpec((BM, HALF), _row_blk_hi),
            *_deg_specs(),
            pl.BlockSpec((1, D), lambda c, i: (0, 0)),
            pl.BlockSpec((D, HALF), lambda c, i: (0, c)),
        ],
        out_specs=pl.BlockSpec((BM, HALF), lambda c, i: (c * NB + i, 0)),
        out_shape=jax.ShapeDtypeStruct((2 * NP, HALF), jnp.float32),
    )(scat, scat, gcat, gcat, degp, degp, b2d, w)


def _tc_cls(scat, gcat, degp, b2d, wcp, bcp):
    return pl.pallas_call(
        _tc_cls_body,
        grid=(NB,),
        in_specs=[
            pl.BlockSpec((BM, HALF), lambda i: (i, 0)),
            pl.BlockSpec((BM, HALF), lambda i: (NB + i, 0)),
            pl.BlockSpec((BM, HALF), lambda i: (i, 0)),
            pl.BlockSpec((BM, HALF), lambda i: (NB + i, 0)),
            pl.BlockSpec((BM, HALF), lambda i: (i, 0)),
            pl.BlockSpec((BM, HALF), lambda i: (NB + i, 0)),
            pl.BlockSpec((1, D), lambda i: (0, 0)),
            pl.BlockSpec((D, HALF), lambda i: (0, 0)),
            pl.BlockSpec((1, HALF), lambda i: (0, 0)),
        ],
        out_specs=pl.BlockSpec((BM, HALF), lambda i: (i, 0)),
        out_shape=jax.ShapeDtypeStruct((NP, HALF), jnp.float32),
    )(scat, scat, gcat, gcat, degp, degp, b2d, wcp, bcp)


def kernel(x, edge_index, W1, b1, W2, b2, W3, b3, Wc, bc):
    src = edge_index[0].astype(jnp.int32)
    dst = edge_index[1].astype(jnp.int32)
    pad = EP - E
    srcp = jnp.concatenate([src, jnp.zeros((pad,), jnp.int32)])
    dstpad = jnp.concatenate([dst, jnp.full((pad,), NP - 1, jnp.int32)])
    srcf = jnp.concatenate([src, src + NP])
    dstp = dstpad.reshape(EROWS, KA)
    ones_ka = jnp.ones((KA, HALF), jnp.float32)
    zeros128 = jnp.zeros((ROWS_PT, HALF), jnp.float32)
    wcp = jnp.zeros((D, HALF), jnp.float32).at[:, :2].set(Wc)
    bcp = jnp.full((1, HALF), -1e30, jnp.float32).at[0, :2].set(bc)
    x_p = jnp.zeros((NP, D), jnp.float32).at[:N].set(x)

    degp = _sc_deg(dstp, ones_ka, zeros128)
    g = _tc_first(x_p, W1, degp)
    s = _sc_agg(g, srcf, dst, zeros128)
    g = _tc_layer(s, g, degp, b1.reshape(1, D), W2)
    s = _sc_agg(g, srcf, dst, zeros128)
    g = _tc_layer(s, g, degp, b2.reshape(1, D), W3)
    s = _sc_agg(g, srcf, dst, zeros128)
    outp = _tc_cls(s, g, degp, b3.reshape(1, D), wcp, bcp)
    return outp[:N, :2]
